# Initial kernel scaffold; baseline (speedup 1.0000x reference)
import functools

import jax
import jax.numpy as jnp
import numpy as np
from jax import lax
from jax.experimental import pallas as pl
from jax.experimental.pallas import tpu as pltpu
from jax.experimental.pallas import tpu_sc as plsc

_N = 10000
_E = 320000
_D = 128

_NC = 2
_NS = 16
_NW = _NC * _NS

_K = 128
_CHUNKS = 80
_EPW = _K * _CHUNKS
_EP = _NW * _EPW
_PAD = _EP - _E
_DUMP = 240
_NACC = _N + _DUMP
_ZPS = _NACC // _NS
_OPS = _N // _NS

_mesh = plsc.VectorSubcoreMesh(core_axis_name="c", subcore_axis_name="s")


def _sc_degree(dstp):
  @functools.partial(
      pl.kernel,
      out_type=jax.ShapeDtypeStruct((_NC, _N), jnp.float32),
      mesh=_mesh,
      scratch_types=[
          pltpu.VMEM((_CHUNKS, _K), jnp.int32),
          pltpu.VMEM((_K,), jnp.float32),
          pltpu.VMEM((_ZPS,), jnp.float32),
          pltpu.VMEM_SHARED((_NACC,), jnp.float32),
          pltpu.SemaphoreType.DMA,
      ],
  )
  def kern(dstp_hbm, deg_hbm, idx_v, ones_v, z_v, acc_sh, sem):
    c = lax.axis_index("c")
    s = lax.axis_index("s")
    w = c * _NS + s
    cp = pltpu.async_copy(dstp_hbm.at[w], idx_v, sem)

    @pl.loop(0, _K // 16)
    def _(j):
      ones_v[pl.ds(j * 16, 16)] = jnp.ones((16,), jnp.float32)

    @pl.loop(0, _ZPS // 16)
    def _(j):
      z_v[pl.ds(j * 16, 16)] = jnp.zeros((16,), jnp.float32)

    pltpu.sync_copy(z_v, acc_sh.at[pl.ds(s * _ZPS, _ZPS)])
    cp.wait()
    plsc.subcore_barrier()

    @pl.loop(0, _CHUNKS)
    def _(i):
      pltpu.sync_copy(ones_v, acc_sh.at[idx_v.at[i]], add=True)

    plsc.subcore_barrier()

    @pl.when(s == 0)
    def _():
      pltpu.sync_copy(acc_sh.at[pl.ds(0, _N)], deg_hbm.at[c])

  return kern(dstp)


def _sc_spmm(hs, srcp, dstp, zrows):
  @functools.partial(
      pl.kernel,
      out_type=jax.ShapeDtypeStruct((_NC, _N, _D), jnp.float32),
      mesh=_mesh,
      scratch_types=[
          pltpu.VMEM((_CHUNKS, _K), jnp.int32),
          pltpu.VMEM((_CHUNKS, _K), jnp.int32),
          pltpu.VMEM((_K, _D), jnp.float32),
          pltpu.VMEM((_K, _D), jnp.float32),
          pltpu.VMEM_SHARED((_NACC, _D), jnp.float32),
          pltpu.SemaphoreType.DMA,
          pltpu.SemaphoreType.DMA,
          pltpu.SemaphoreType.DMA,
      ],
  )
  def kern(hs_hbm, srcp_hbm, dstp_hbm, z_hbm, out_hbm,
           src_v, dst_v, b0, b1, acc_sh, sem_i, sem0, sem1):
    c = lax.axis_index("c")
    s = lax.axis_index("s")
    w = c * _NS + s
    cps = pltpu.async_copy(srcp_hbm.at[w], src_v, sem_i)
    cpd = pltpu.async_copy(dstp_hbm.at[w], dst_v, sem_i)
    pltpu.sync_copy(z_hbm.at[pl.ds(s * _ZPS, _ZPS)],
                    acc_sh.at[pl.ds(s * _ZPS, _ZPS)])
    cps.wait()
    cpd.wait()
    plsc.subcore_barrier()

    pltpu.async_copy(hs_hbm.at[src_v.at[0]], b0, sem0)

    @pl.loop(0, _CHUNKS // 2 - 1)
    def _(i):
      j = 2 * i
      pltpu.make_async_copy(hs_hbm.at[src_v.at[j]], b0, sem0).wait()
      pltpu.async_copy(hs_hbm.at[src_v.at[j + 1]], b1, sem1)
      pltpu.sync_copy(b0, acc_sh.at[dst_v.at[j]], add=True)
      pltpu.make_async_copy(hs_hbm.at[src_v.at[j + 1]], b1, sem1).wait()
      pltpu.async_copy(hs_hbm.at[src_v.at[j + 2]], b0, sem0)
      pltpu.sync_copy(b1, acc_sh.at[dst_v.at[j + 1]], add=True)

    pltpu.make_async_copy(hs_hbm.at[src_v.at[_CHUNKS - 2]], b0, sem0).wait()
    pltpu.async_copy(hs_hbm.at[src_v.at[_CHUNKS - 1]], b1, sem1)
    pltpu.sync_copy(b0, acc_sh.at[dst_v.at[_CHUNKS - 2]], add=True)
    pltpu.make_async_copy(hs_hbm.at[src_v.at[_CHUNKS - 1]], b1, sem1).wait()
    pltpu.sync_copy(b1, acc_sh.at[dst_v.at[_CHUNKS - 1]], add=True)

    plsc.subcore_barrier()
    pltpu.sync_copy(acc_sh.at[pl.ds(s * _OPS, _OPS)],
                    out_hbm.at[c].at[pl.ds(s * _OPS, _OPS)])

  return kern(hs, srcp, dstp, zrows)


_SQRT_HALF = float(1.0 / np.sqrt(2.0))


def _ln(x, g, b):
  m = jnp.mean(x, axis=-1, keepdims=True)
  xc = x - m
  v = jnp.mean(xc * xc, axis=-1, keepdims=True)
  return xc * lax.rsqrt(v + 1e-5) * g[None, :] + b[None, :]


def _tc_pre(x, g1, beta1, W1):
  def body(x_ref, g_ref, b_ref, w_ref, h_ref):
    xn = _ln(x_ref[...], g_ref[...], b_ref[...])
    h_ref[...] = jnp.dot(xn, w_ref[...], preferred_element_type=jnp.float32)

  return pl.pallas_call(
      body, out_shape=jax.ShapeDtypeStruct((_N, _D), jnp.float32),
  )(x, g1, beta1, W1)


def _tc_scale1(degT, h1, b1):
  def body(d_ref, h_ref, b_ref, hs_ref, s1_ref, di_ref):
    deg = d_ref[:, 0:1] + d_ref[:, 1:2] + 1.0
    di = lax.rsqrt(deg)
    h = h_ref[...]
    hs_ref[...] = h * di
    s1_ref[...] = h * (di * di) + b_ref[...][None, :]
    di_ref[...] = di

  return pl.pallas_call(
      body,
      out_shape=(
          jax.ShapeDtypeStruct((_N, _D), jnp.float32),
          jax.ShapeDtypeStruct((_N, _D), jnp.float32),
          jax.ShapeDtypeStruct((_N, 1), jnp.float32),
      ),
  )(degT, h1, b1)


def _tc_mid(acc, self1, dinv, g4, beta4, W4, b4):
  def body(a_ref, s1_ref, di_ref, g_ref, be_ref, w_ref, b_ref,
           hs_ref, s2_ref):
    di = di_ref[...]
    h1 = (a_ref[0] + a_ref[1]) * di + s1_ref[...]
    gel = 0.5 * h1 * (1.0 + lax.erf(h1 * _SQRT_HALF))
    gn = _ln(gel, g_ref[...], be_ref[...])
    h2 = jnp.dot(gn, w_ref[...], preferred_element_type=jnp.float32)
    hs_ref[...] = h2 * di
    s2_ref[...] = h2 * (di * di) + b_ref[...][None, :]

  return pl.pallas_call(
      body,
      out_shape=(
          jax.ShapeDtypeStruct((_N, _D), jnp.float32),
          jax.ShapeDtypeStruct((_N, _D), jnp.float32),
      ),
  )(acc, self1, dinv, g4, beta4, W4, b4)


def _tc_post(acc, self2, dinv):
  def body(a_ref, s2_ref, di_ref, o_ref):
    o_ref[...] = (a_ref[0] + a_ref[1]) * di_ref[...] + s2_ref[...]

  return pl.pallas_call(
      body, out_shape=jax.ShapeDtypeStruct((_N, _D), jnp.float32),
  )(acc, self2, dinv)


def kernel(x, edge_index, W1, b1, W4, b4, g1, beta1, g4, beta4):
  src = edge_index[0]
  dst = edge_index[1]
  pi = jnp.arange(_PAD, dtype=jnp.int32)
  srcp = jnp.concatenate([src, pi % _N]).reshape(_NW, _CHUNKS, _K)
  dstp = jnp.concatenate([dst, _N + pi % _DUMP]).reshape(_NW, _CHUNKS, _K)
  zrows = jnp.zeros((_NACC, _D), jnp.float32)

  deg_p = _sc_degree(dstp)
  h1 = _tc_pre(x, g1, beta1, W1)
  hs1, self1, dinv = _tc_scale1(deg_p.T, h1, b1)
  acc1 = _sc_spmm(hs1, srcp, dstp, zrows)
  hs2, self2 = _tc_mid(acc1, self1, dinv, g4, beta4, W4, b4)
  acc2 = _sc_spmm(hs2, srcp, dstp, zrows)
  return _tc_post(acc2, self2, dinv)


# speedup vs baseline: 26.2906x; 26.2906x over previous
import functools

import jax
import jax.numpy as jnp
import numpy as np
from jax import lax
from jax.experimental import pallas as pl
from jax.experimental.pallas import tpu as pltpu
from jax.experimental.pallas import tpu_sc as plsc

_N = 10000
_E = 320000
_D = 128

_NC = 2
_NS = 16
_NW = _NC * _NS

_K = 128
_CHUNKS = 80
_EPW = _K * _CHUNKS
_EP = _NW * _EPW
_PAD = _EP - _E
_DUMP = 240
_NACC = _N + _DUMP
_ZPS = _NACC // _NS
_OPS = _N // _NS

_mesh = plsc.VectorSubcoreMesh(core_axis_name="c", subcore_axis_name="s")


def _sc_degree(ep):
  @functools.partial(
      pl.kernel,
      out_type=jax.ShapeDtypeStruct((_NC, _NACC), jnp.float32),
      mesh=_mesh,
      scratch_types=[
          pltpu.VMEM((_CHUNKS, 2, _K), jnp.int32),
          pltpu.VMEM((_K,), jnp.float32),
          pltpu.VMEM((_ZPS,), jnp.float32),
          pltpu.VMEM_SHARED((_NACC,), jnp.float32),
          pltpu.SemaphoreType.DMA,
      ],
  )
  def kern(ep_hbm, deg_hbm, idx_v, ones_v, z_v, acc_sh, sem):
    c = lax.axis_index("c")
    s = lax.axis_index("s")
    w = c * _NS + s
    cp = pltpu.async_copy(ep_hbm.at[w], idx_v, sem)

    @pl.loop(0, _K // 16)
    def _(j):
      ones_v[pl.ds(j * 16, 16)] = jnp.ones((16,), jnp.float32)

    @pl.loop(0, _ZPS // 16)
    def _(j):
      z_v[pl.ds(j * 16, 16)] = jnp.zeros((16,), jnp.float32)

    pltpu.sync_copy(z_v, acc_sh.at[pl.ds(s * _ZPS, _ZPS)])
    cp.wait()
    plsc.subcore_barrier()

    @pl.loop(0, _CHUNKS)
    def _(i):
      pltpu.sync_copy(ones_v, acc_sh.at[idx_v.at[i].at[1]], add=True)

    plsc.subcore_barrier()

    pltpu.sync_copy(acc_sh.at[pl.ds(s * _ZPS, _ZPS)],
                    deg_hbm.at[c].at[pl.ds(s * _ZPS, _ZPS)])

  return kern(ep)


def _sc_spmm(hs, ep, zrows):
  @functools.partial(
      pl.kernel,
      out_type=jax.ShapeDtypeStruct((_NC, _NACC, _D), jnp.float32),
      mesh=_mesh,
      scratch_types=[
          pltpu.VMEM((2, _K), jnp.int32),
          pltpu.VMEM((2, _K), jnp.int32),
          pltpu.VMEM((_K, _D), jnp.float32),
          pltpu.VMEM((_K, _D), jnp.float32),
          pltpu.VMEM_SHARED((_NACC, _D), jnp.float32),
          pltpu.SemaphoreType.DMA,
          pltpu.SemaphoreType.DMA,
          pltpu.SemaphoreType.DMA,
          pltpu.SemaphoreType.DMA,
      ],
  )
  def kern(hs_hbm, ep_hbm, z_hbm, out_hbm,
           ib0, ib1, b0, b1, acc_sh, semi0, semi1, sem0, sem1):
    c = lax.axis_index("c")
    s = lax.axis_index("s")
    w = c * _NS + s
    epw = ep_hbm.at[w]
    pltpu.sync_copy(z_hbm.at[pl.ds(s * _ZPS, _ZPS)],
                    acc_sh.at[pl.ds(s * _ZPS, _ZPS)])
    plsc.subcore_barrier()

    pltpu.sync_copy(epw.at[0], ib0)
    pltpu.async_copy(hs_hbm.at[ib0.at[0]], b0, sem0)
    pltpu.async_copy(epw.at[1], ib1, semi1)

    @pl.loop(0, _CHUNKS // 2 - 1)
    def _(i):
      j = 2 * i
      pltpu.make_async_copy(hs_hbm.at[ib0.at[0]], b0, sem0).wait()
      pltpu.make_async_copy(epw.at[j + 1], ib1, semi1).wait()
      pltpu.async_copy(hs_hbm.at[ib1.at[0]], b1, sem1)
      pltpu.sync_copy(b0, acc_sh.at[ib0.at[1]], add=True)
      pltpu.async_copy(epw.at[j + 2], ib0, semi0)
      pltpu.make_async_copy(hs_hbm.at[ib1.at[0]], b1, sem1).wait()
      pltpu.make_async_copy(epw.at[j + 2], ib0, semi0).wait()
      pltpu.async_copy(hs_hbm.at[ib0.at[0]], b0, sem0)
      pltpu.sync_copy(b1, acc_sh.at[ib1.at[1]], add=True)
      pltpu.async_copy(epw.at[j + 3], ib1, semi1)

    pltpu.make_async_copy(hs_hbm.at[ib0.at[0]], b0, sem0).wait()
    pltpu.make_async_copy(epw.at[_CHUNKS - 1], ib1, semi1).wait()
    pltpu.async_copy(hs_hbm.at[ib1.at[0]], b1, sem1)
    pltpu.sync_copy(b0, acc_sh.at[ib0.at[1]], add=True)
    pltpu.make_async_copy(hs_hbm.at[ib1.at[0]], b1, sem1).wait()
    pltpu.sync_copy(b1, acc_sh.at[ib1.at[1]], add=True)

    plsc.subcore_barrier()
    pltpu.sync_copy(acc_sh.at[pl.ds(s * _ZPS, _ZPS)],
                    out_hbm.at[c].at[pl.ds(s * _ZPS, _ZPS)])

  return kern(hs, ep, zrows)


_SQRT_HALF = float(1.0 / np.sqrt(2.0))


def _ln(x, g, b):
  m = jnp.mean(x, axis=-1, keepdims=True)
  xc = x - m
  v = jnp.mean(xc * xc, axis=-1, keepdims=True)
  return xc * lax.rsqrt(v + 1e-5) * g[None, :] + b[None, :]


def _tc_pre(x, g1, beta1, W1):
  def body(x_ref, g_ref, b_ref, w_ref, h_ref):
    xn = _ln(x_ref[...], g_ref[...], b_ref[...])
    h_ref[...] = jnp.dot(xn, w_ref[...], preferred_element_type=jnp.float32)

  return pl.pallas_call(
      body, out_shape=jax.ShapeDtypeStruct((_N, _D), jnp.float32),
  )(x, g1, beta1, W1)


def _tc_scale1(degT, h1, b1):
  def body(d_ref, h_ref, b_ref, hs_ref, s1_ref, di_ref):
    dd = d_ref[...][: _N]
    deg = dd[:, 0:1] + dd[:, 1:2] + 1.0
    di = lax.rsqrt(deg)
    h = h_ref[...]
    hs_ref[...] = h * di
    s1_ref[...] = h * (di * di) + b_ref[...][None, :]
    di_ref[...] = di

  return pl.pallas_call(
      body,
      out_shape=(
          jax.ShapeDtypeStruct((_N, _D), jnp.float32),
          jax.ShapeDtypeStruct((_N, _D), jnp.float32),
          jax.ShapeDtypeStruct((_N, 1), jnp.float32),
      ),
  )(degT, h1, b1)


def _tc_mid(acc, self1, dinv, g4, beta4, W4, b4):
  def body(a_ref, s1_ref, di_ref, g_ref, be_ref, w_ref, b_ref,
           hs_ref, s2_ref):
    di = di_ref[...]
    a = a_ref[...]
    h1 = (a[0, : _N] + a[1, : _N]) * di + s1_ref[...]
    gel = 0.5 * h1 * (1.0 + lax.erf(h1 * _SQRT_HALF))
    gn = _ln(gel, g_ref[...], be_ref[...])
    h2 = jnp.dot(gn, w_ref[...], preferred_element_type=jnp.float32)
    hs_ref[...] = h2 * di
    s2_ref[...] = h2 * (di * di) + b_ref[...][None, :]

  return pl.pallas_call(
      body,
      out_shape=(
          jax.ShapeDtypeStruct((_N, _D), jnp.float32),
          jax.ShapeDtypeStruct((_N, _D), jnp.float32),
      ),
  )(acc, self1, dinv, g4, beta4, W4, b4)


def _tc_post(acc, self2, dinv):
  def body(a_ref, s2_ref, di_ref, o_ref):
    a = a_ref[...]
    o_ref[...] = (a[0, : _N] + a[1, : _N]) * di_ref[...] + s2_ref[...]

  return pl.pallas_call(
      body, out_shape=jax.ShapeDtypeStruct((_N, _D), jnp.float32),
  )(acc, self2, dinv)


def kernel(x, edge_index, W1, b1, W4, b4, g1, beta1, g4, beta4):
  src = edge_index[0]
  dst = edge_index[1]
  pi = jnp.arange(_PAD, dtype=jnp.int32)
  srcp = jnp.concatenate([src, pi % _N]).reshape(_NW, _CHUNKS, 1, _K)
  dstp = jnp.concatenate([dst, _N + pi % _DUMP]).reshape(_NW, _CHUNKS, 1, _K)
  ep = jnp.concatenate([srcp, dstp], axis=2)
  zrows = jnp.zeros((_NACC, _D), jnp.float32)

  deg_p = _sc_degree(ep)
  h1 = _tc_pre(x, g1, beta1, W1)
  hs1, self1, dinv = _tc_scale1(deg_p.T, h1, b1)
  acc1 = _sc_spmm(hs1, ep, zrows)
  hs2, self2 = _tc_mid(acc1, self1, dinv, g4, beta4, W4, b4)
  acc2 = _sc_spmm(hs2, ep, zrows)
  return _tc_post(acc2, self2, dinv)


# speedup vs baseline: 28.1949x; 1.0724x over previous
import functools

import jax
import jax.numpy as jnp
import numpy as np
from jax import lax
from jax.experimental import pallas as pl
from jax.experimental.pallas import tpu as pltpu
from jax.experimental.pallas import tpu_sc as plsc

_N = 10000
_E = 320000
_D = 128

_NC = 2
_NS = 16
_NW = _NC * _NS

_K = 64
_SUB = 4
_SC_CHUNKS = 40
_EPW = _K * _SUB * _SC_CHUNKS
_EP = _NW * _EPW
_PAD = _EP - _E
_DUMP = 240
_NACC = _N + _DUMP
_ZPS = _NACC // _NS

_mesh = plsc.VectorSubcoreMesh(core_axis_name="c", subcore_axis_name="s")


def _sc_degree(ep):
  @functools.partial(
      pl.kernel,
      out_type=jax.ShapeDtypeStruct((_NC, _NACC), jnp.float32),
      mesh=_mesh,
      scratch_types=[
          pltpu.VMEM((_SC_CHUNKS, 2 * _SUB, _K), jnp.int32),
          pltpu.VMEM((_K,), jnp.float32),
          pltpu.VMEM((_ZPS,), jnp.float32),
          pltpu.VMEM_SHARED((_NACC,), jnp.float32),
          pltpu.SemaphoreType.DMA,
      ],
  )
  def kern(ep_hbm, deg_hbm, idx_v, ones_v, z_v, acc_sh, sem):
    c = lax.axis_index("c")
    s = lax.axis_index("s")
    w = c * _NS + s
    cp = pltpu.async_copy(ep_hbm.at[w], idx_v, sem)

    @pl.loop(0, _K // 16)
    def _(j):
      ones_v[pl.ds(j * 16, 16)] = jnp.ones((16,), jnp.float32)

    @pl.loop(0, _ZPS // 16)
    def _(j):
      z_v[pl.ds(j * 16, 16)] = jnp.zeros((16,), jnp.float32)

    pltpu.sync_copy(z_v, acc_sh.at[pl.ds(s * _ZPS, _ZPS)])
    cp.wait()
    plsc.subcore_barrier()

    @pl.loop(0, _SC_CHUNKS)
    def _(i):
      for t in range(_SUB):
        pltpu.sync_copy(ones_v, acc_sh.at[idx_v.at[i].at[_SUB + t]], add=True)

    plsc.subcore_barrier()

    pltpu.sync_copy(acc_sh.at[pl.ds(s * _ZPS, _ZPS)],
                    deg_hbm.at[c].at[pl.ds(s * _ZPS, _ZPS)])

  return kern(ep)


def _sc_spmm(hs, ep, zrows):
  @functools.partial(
      pl.kernel,
      out_type=jax.ShapeDtypeStruct((_NC, _NACC, _D), jnp.float32),
      mesh=_mesh,
      scratch_types=[
          pltpu.VMEM((2 * _SUB, _K), jnp.int32),
          pltpu.VMEM((2 * _SUB, _K), jnp.int32),
          pltpu.VMEM((_K, _D), jnp.float32),
          pltpu.VMEM((_K, _D), jnp.float32),
          pltpu.VMEM((_K, _D), jnp.float32),
          pltpu.VMEM((_K, _D), jnp.float32),
          pltpu.VMEM_SHARED((_NACC, _D), jnp.float32),
          pltpu.SemaphoreType.DMA,
          pltpu.SemaphoreType.DMA,
          pltpu.SemaphoreType.DMA,
          pltpu.SemaphoreType.DMA,
          pltpu.SemaphoreType.DMA,
          pltpu.SemaphoreType.DMA,
      ],
  )
  def kern(hs_hbm, ep_hbm, z_hbm, out_hbm,
           q0, q1, b0, b1, b2, b3, acc_sh,
           semq0, semq1, sg0, sg1, sg2, sg3):
    bufs = (b0, b1, b2, b3)
    sgs = (sg0, sg1, sg2, sg3)
    c = lax.axis_index("c")
    s = lax.axis_index("s")
    w = c * _NS + s
    epw = ep_hbm.at[w]
    pltpu.sync_copy(z_hbm.at[pl.ds(s * _ZPS, _ZPS)],
                    acc_sh.at[pl.ds(s * _ZPS, _ZPS)])
    plsc.subcore_barrier()

    def gathers(q):
      for t in range(_SUB):
        pltpu.async_copy(hs_hbm.at[q.at[t]], bufs[t], sgs[t])

    def process(qcur, qnext):
      for t in range(_SUB):
        pltpu.make_async_copy(hs_hbm.at[qcur.at[t]], bufs[t], sgs[t]).wait()
        pltpu.sync_copy(bufs[t], acc_sh.at[qcur.at[_SUB + t]], add=True)
        pltpu.async_copy(hs_hbm.at[qnext.at[t]], bufs[t], sgs[t])

    def drain(qcur):
      for t in range(_SUB):
        pltpu.make_async_copy(hs_hbm.at[qcur.at[t]], bufs[t], sgs[t]).wait()
        pltpu.sync_copy(bufs[t], acc_sh.at[qcur.at[_SUB + t]], add=True)

    pltpu.sync_copy(epw.at[0], q0)
    gathers(q0)
    pltpu.async_copy(epw.at[1], q1, semq1)

    @pl.loop(0, _SC_CHUNKS // 2 - 1)
    def _(i):
      a = 2 * i
      pltpu.make_async_copy(epw.at[a + 1], q1, semq1).wait()
      process(q0, q1)
      pltpu.async_copy(epw.at[a + 2], q0, semq0)
      pltpu.make_async_copy(epw.at[a + 2], q0, semq0).wait()
      process(q1, q0)
      pltpu.async_copy(epw.at[a + 3], q1, semq1)

    pltpu.make_async_copy(epw.at[_SC_CHUNKS - 1], q1, semq1).wait()
    process(q0, q1)
    drain(q1)

    plsc.subcore_barrier()
    pltpu.sync_copy(acc_sh.at[pl.ds(s * _ZPS, _ZPS)],
                    out_hbm.at[c].at[pl.ds(s * _ZPS, _ZPS)])

  return kern(hs, ep, zrows)


_SQRT_HALF = float(1.0 / np.sqrt(2.0))


def _ln(x, g, b):
  m = jnp.mean(x, axis=-1, keepdims=True)
  xc = x - m
  v = jnp.mean(xc * xc, axis=-1, keepdims=True)
  return xc * lax.rsqrt(v + 1e-5) * g[None, :] + b[None, :]


def _tc_pre(x, g1, beta1, W1):
  def body(x_ref, g_ref, b_ref, w_ref, h_ref):
    xn = _ln(x_ref[...], g_ref[...], b_ref[...])
    h_ref[...] = jnp.dot(xn, w_ref[...], preferred_element_type=jnp.float32)

  return pl.pallas_call(
      body, out_shape=jax.ShapeDtypeStruct((_N, _D), jnp.float32),
  )(x, g1, beta1, W1)


def _tc_scale1(degT, h1, b1):
  def body(d_ref, h_ref, b_ref, hs_ref, s1_ref, di_ref):
    dd = d_ref[...][: _N]
    deg = dd[:, 0:1] + dd[:, 1:2] + 1.0
    di = lax.rsqrt(deg)
    h = h_ref[...]
    hs_ref[...] = h * di
    s1_ref[...] = h * (di * di) + b_ref[...][None, :]
    di_ref[...] = di

  return pl.pallas_call(
      body,
      out_shape=(
          jax.ShapeDtypeStruct((_N, _D), jnp.float32),
          jax.ShapeDtypeStruct((_N, _D), jnp.float32),
          jax.ShapeDtypeStruct((_N, 1), jnp.float32),
      ),
  )(degT, h1, b1)


def _tc_mid(acc, self1, dinv, g4, beta4, W4, b4):
  def body(a_ref, s1_ref, di_ref, g_ref, be_ref, w_ref, b_ref,
           hs_ref, s2_ref):
    di = di_ref[...]
    a = a_ref[...]
    h1 = (a[0, : _N] + a[1, : _N]) * di + s1_ref[...]
    gel = 0.5 * h1 * (1.0 + lax.erf(h1 * _SQRT_HALF))
    gn = _ln(gel, g_ref[...], be_ref[...])
    h2 = jnp.dot(gn, w_ref[...], preferred_element_type=jnp.float32)
    hs_ref[...] = h2 * di
    s2_ref[...] = h2 * (di * di) + b_ref[...][None, :]

  return pl.pallas_call(
      body,
      out_shape=(
          jax.ShapeDtypeStruct((_N, _D), jnp.float32),
          jax.ShapeDtypeStruct((_N, _D), jnp.float32),
      ),
  )(acc, self1, dinv, g4, beta4, W4, b4)


def _tc_post(acc, self2, dinv):
  def body(a_ref, s2_ref, di_ref, o_ref):
    a = a_ref[...]
    o_ref[...] = (a[0, : _N] + a[1, : _N]) * di_ref[...] + s2_ref[...]

  return pl.pallas_call(
      body, out_shape=jax.ShapeDtypeStruct((_N, _D), jnp.float32),
  )(acc, self2, dinv)


def kernel(x, edge_index, W1, b1, W4, b4, g1, beta1, g4, beta4):
  src = edge_index[0]
  dst = edge_index[1]
  pi = jnp.arange(_PAD, dtype=jnp.int32)
  srcp = jnp.concatenate([src, pi % _N]).reshape(_NW, _SC_CHUNKS, _SUB, _K)
  dstp = jnp.concatenate([dst, _N + pi % _DUMP]).reshape(
      _NW, _SC_CHUNKS, _SUB, _K)
  ep = jnp.concatenate([srcp, dstp], axis=2)
  zrows = jnp.zeros((_NACC, _D), jnp.float32)

  deg_p = _sc_degree(ep)
  h1 = _tc_pre(x, g1, beta1, W1)
  hs1, self1, dinv = _tc_scale1(deg_p.T, h1, b1)
  acc1 = _sc_spmm(hs1, ep, zrows)
  hs2, self2 = _tc_mid(acc1, self1, dinv, g4, beta4, W4, b4)
  acc2 = _sc_spmm(hs2, ep, zrows)
  return _tc_post(acc2, self2, dinv)
